# baseline (speedup 1.0000x reference)
import functools

import jax
import jax.numpy as jnp
from jax import lax
from jax.experimental import pallas as pl
from jax.experimental.pallas import tpu as pltpu
from jax.experimental.pallas import tpu_sc as plsc

_NW = 32
_IDXCH = 64
_GRP = 4
_NGRP = _IDXCH // _GRP
_ROW = 104


@jax.jit
def _lookup(token_ids, table2):
    nb, seq = token_ids.shape
    dim = 32
    nb_per_w = nb // _NW
    nchunks = nb_per_w // _IDXCH

    @functools.partial(
        pl.kernel,
        mesh=plsc.VectorSubcoreMesh(core_axis_name="c", subcore_axis_name="s"),
        out_type=jax.ShapeDtypeStruct((nb, dim, seq), jnp.float32),
        scratch_types=[
            pltpu.VMEM((_IDXCH, seq), jnp.int32),
            pltpu.VMEM((2 * _NGRP, _ROW), jnp.int32),
            pltpu.VMEM((2 * seq, 128), jnp.float32),
            pltpu.VMEM((2 * seq, 128), jnp.float32),
            pltpu.VMEM((2 * seq, 128), jnp.float32),
            pltpu.VMEM((2 * seq, 128), jnp.float32),
            pltpu.VMEM((2 * _GRP, dim, seq), jnp.float32),
            pltpu.SemaphoreType.DMA,
            pltpu.SemaphoreType.DMA,
            pltpu.SemaphoreType.DMA,
        ],
        compiler_params=pltpu.CompilerParams(needs_layout_passes=False),
    )
    def k(idx_hbm, table_hbm, out_hbm, idx_v, blk_v,
          buf_a0, buf_a1, buf_b0, buf_b1, ov,
          sem_a, sem_b, sem_o):
        wid = lax.axis_index("s") * 2 + lax.axis_index("c")
        b_base = wid * nb_per_w

        def fire(gi, buf0, buf1, sem):
            c0 = pltpu.async_copy(
                table_hbm.at[blk_v.at[2 * gi, pl.ds(0, 2 * seq)]], buf0, sem)
            c1 = pltpu.async_copy(
                table_hbm.at[blk_v.at[2 * gi + 1, pl.ds(0, 2 * seq)]], buf1, sem)
            return c0, c1

        def extract(gi, buf0, buf1, half):
            r0 = gi * _GRP
            for lr in range(_GRP):
                buf = buf0 if lr < 2 else buf1
                tbase = seq * (lr % 2)
                for t0 in (0, 16, 32, 34):
                    offv = (idx_v[r0 + lr, pl.ds(t0, 16)] & 3) * dim
                    for lane in range(14 if t0 == 34 else 0, 16):
                        s = t0 + lane
                        off = offv[lane]
                        tloc = tbase + s
                        for h in (0, 1):
                            plsc.store_scatter(
                                ov.at[half * _GRP + lr],
                                [lax.iota(jnp.int32, 16) + 16 * h,
                                 jnp.full((16,), s, jnp.int32)],
                                buf[tloc, pl.ds(off + 16 * h, 16)])

        def chunk(ci, carry):
            b0 = pl.multiple_of(b_base + ci * _IDXCH, _IDXCH)
            pltpu.sync_copy(idx_hbm.at[pl.ds(b0, _IDXCH)], idx_v)
            for r in range(_IDXCH):
                gi, p = r // _GRP, r % _GRP
                row, dbase = 2 * gi + p // 2, seq * (p % 2)
                for off in (0, 16, 32, 34):
                    blk_v[row, pl.ds(dbase + off, 16)] = (
                        idx_v[r, pl.ds(off, 16)] >> 2)

            def group_pair(kk, c2):
                ga = kk * 2
                a0, a1 = fire(ga, buf_a0, buf_a1, sem_a)
                b0_, b1_ = fire(ga + 1, buf_b0, buf_b1, sem_b)
                a0.wait()
                a1.wait()
                extract(ga, buf_a0, buf_a1, 0)
                b0_.wait()
                b1_.wait()
                extract(ga + 1, buf_b0, buf_b1, 1)
                pltpu.sync_copy(
                    ov, out_hbm.at[pl.ds(
                        pl.multiple_of(b0 + kk * 2 * _GRP, 2 * _GRP),
                        2 * _GRP)])
                return c2

            lax.fori_loop(0, _NGRP // 2, group_pair, 0)
            return carry

        lax.fori_loop(0, nchunks, chunk, 0)

    return k(token_ids, table2)


def kernel(token_ids, weight):
    table2 = weight.reshape(weight.shape[0] // 4, 128)
    out = _lookup(token_ids.astype(jnp.int32), table2)
    return out.transpose(0, 2, 1)


# speedup vs baseline: 1.0877x; 1.0877x over previous
import functools

import jax
import jax.numpy as jnp
from jax import lax
from jax.experimental import pallas as pl
from jax.experimental.pallas import tpu as pltpu
from jax.experimental.pallas import tpu_sc as plsc

_NW = 32
_IDXCH = 64
_GRP = 4
_NGRP = _IDXCH // _GRP
_ROW = 104


@jax.jit
def _lookup(token_ids, table2):
    nb, seq = token_ids.shape
    dim = 32
    nb_per_w = nb // _NW
    nchunks = nb_per_w // _IDXCH

    @functools.partial(
        pl.kernel,
        mesh=plsc.VectorSubcoreMesh(core_axis_name="c", subcore_axis_name="s"),
        out_type=jax.ShapeDtypeStruct((nb, seq, dim), jnp.float32),
        scratch_types=[
            pltpu.VMEM((_IDXCH, seq), jnp.int32),
            pltpu.VMEM((2 * _NGRP, _ROW), jnp.int32),
            pltpu.VMEM((2 * seq, 128), jnp.float32),
            pltpu.VMEM((2 * seq, 128), jnp.float32),
            pltpu.VMEM((2 * seq, 128), jnp.float32),
            pltpu.VMEM((2 * seq, 128), jnp.float32),
            pltpu.VMEM((2 * _GRP, seq, dim), jnp.float32),
            pltpu.SemaphoreType.DMA,
            pltpu.SemaphoreType.DMA,
            pltpu.SemaphoreType.DMA,
        ],
    )
    def k(idx_hbm, table_hbm, out_hbm, idx_v, blk_v,
          buf_a0, buf_a1, buf_b0, buf_b1, ov,
          sem_a, sem_b, sem_o):
        wid = lax.axis_index("s") * 2 + lax.axis_index("c")
        b_base = wid * nb_per_w

        def fire(gi, buf0, buf1, sem):
            c0 = pltpu.async_copy(
                table_hbm.at[blk_v.at[2 * gi, pl.ds(0, 2 * seq)]], buf0, sem)
            c1 = pltpu.async_copy(
                table_hbm.at[blk_v.at[2 * gi + 1, pl.ds(0, 2 * seq)]], buf1, sem)
            return c0, c1

        def extract(gi, buf0, buf1, half):
            r0 = gi * _GRP
            for lr in range(_GRP):
                buf = buf0 if lr < 2 else buf1
                tbase = seq * (lr % 2)
                for t0 in (0, 16, 32, 34):
                    offv = (idx_v[r0 + lr, pl.ds(t0, 16)] & 3) * dim
                    for lane in range(14 if t0 == 34 else 0, 16):
                        s = t0 + lane
                        off = offv[lane]
                        tloc = tbase + s
                        for h in (0, 1):
                            ov[half * _GRP + lr, s, pl.ds(16 * h, 16)] = (
                                buf[tloc, pl.ds(off + 16 * h, 16)])

        def chunk(ci, carry):
            b0 = pl.multiple_of(b_base + ci * _IDXCH, _IDXCH)
            pltpu.sync_copy(idx_hbm.at[pl.ds(b0, _IDXCH)], idx_v)
            for r in range(_IDXCH):
                gi, p = r // _GRP, r % _GRP
                row, dbase = 2 * gi + p // 2, seq * (p % 2)
                for off in (0, 16, 32, 34):
                    blk_v[row, pl.ds(dbase + off, 16)] = (
                        idx_v[r, pl.ds(off, 16)] >> 2)

            def group_pair(kk, c2):
                ga = kk * 2
                a0, a1 = fire(ga, buf_a0, buf_a1, sem_a)
                b0_, b1_ = fire(ga + 1, buf_b0, buf_b1, sem_b)
                a0.wait()
                a1.wait()
                extract(ga, buf_a0, buf_a1, 0)
                b0_.wait()
                b1_.wait()
                extract(ga + 1, buf_b0, buf_b1, 1)
                pltpu.sync_copy(
                    ov, out_hbm.at[pl.ds(
                        pl.multiple_of(b0 + kk * 2 * _GRP, 2 * _GRP),
                        2 * _GRP)])
                return c2

            lax.fori_loop(0, _NGRP // 2, group_pair, 0)
            return carry

        lax.fori_loop(0, nchunks, chunk, 0)

    return k(token_ids, table2)


def kernel(token_ids, weight):
    nblk = weight.shape[0] // 4
    table2 = (weight.T.reshape(32, nblk, 4)
              .transpose(1, 2, 0).reshape(nblk, 128))
    return _lookup(token_ids.astype(jnp.int32), table2)


# speedup vs baseline: 1.2079x; 1.1105x over previous
import functools

import jax
import jax.numpy as jnp
from jax import lax
from jax.experimental import pallas as pl
from jax.experimental.pallas import tpu as pltpu
from jax.experimental.pallas import tpu_sc as plsc

_NW = 32
_IDXCH = 64
_GRP = 4
_NGRP = _IDXCH // _GRP
_ROW = 104


@jax.jit
def _lookup(token_ids, table2):
    nb, seq = token_ids.shape
    dim = 32
    nb_per_w = nb // _NW
    nchunks = nb_per_w // _IDXCH

    @functools.partial(
        pl.kernel,
        mesh=plsc.VectorSubcoreMesh(core_axis_name="c", subcore_axis_name="s"),
        out_type=jax.ShapeDtypeStruct((nb, seq, dim), jnp.float32),
        scratch_types=[
            pltpu.VMEM((_IDXCH, seq), jnp.int32),
            pltpu.VMEM((2 * _NGRP, _ROW), jnp.int32),
            pltpu.VMEM((_ROW, 128), jnp.float32),
            pltpu.VMEM((_ROW, 128), jnp.float32),
            pltpu.VMEM((_ROW, 128), jnp.float32),
            pltpu.VMEM((_ROW, 128), jnp.float32),
            pltpu.VMEM((2 * _GRP, seq, dim), jnp.float32),
            pltpu.SemaphoreType.DMA,
            pltpu.SemaphoreType.DMA,
            pltpu.SemaphoreType.DMA,
        ],
    )
    def k(idx_hbm, table_hbm, out_hbm, idx_v, blk_v,
          buf_a0, buf_a1, buf_b0, buf_b1, ov,
          sem_a, sem_b, sem_o):
        wid = lax.axis_index("s") * 2 + lax.axis_index("c")
        b_base = wid * nb_per_w

        def fire(gi, buf0, buf1, sem):
            pltpu.async_copy(table_hbm.at[blk_v.at[2 * gi]], buf0, sem)
            pltpu.async_copy(table_hbm.at[blk_v.at[2 * gi + 1]], buf1, sem)

        def drain(buf0, buf1, sem):
            pltpu.make_async_copy(
                table_hbm.at[pl.ds(0, _ROW)], buf0, sem).wait()
            pltpu.make_async_copy(
                table_hbm.at[pl.ds(0, _ROW)], buf1, sem).wait()

        def drain_out(b0):
            pltpu.make_async_copy(
                out_hbm.at[pl.ds(b0, 2 * _GRP)], ov, sem_o).wait()

        def extract(gi, buf0, buf1, half):
            r0 = gi * _GRP
            for lr in range(_GRP):
                buf = buf0 if lr < 2 else buf1
                tbase = seq * (lr % 2)
                for t0 in (0, 16, 32, 34):
                    offv = (idx_v[r0 + lr, pl.ds(t0, 16)] & 3) * dim
                    for lane in range(14 if t0 == 34 else 0, 16):
                        s = t0 + lane
                        off = offv[lane]
                        tloc = tbase + s
                        for h in (0, 1):
                            ov[half * _GRP + lr, s, pl.ds(16 * h, 16)] = (
                                buf[tloc, pl.ds(off + 16 * h, 16)])

        def chunk(ci, carry):
            b0 = pl.multiple_of(b_base + ci * _IDXCH, _IDXCH)
            pltpu.sync_copy(idx_hbm.at[pl.ds(b0, _IDXCH)], idx_v)
            for r in range(_IDXCH):
                gi, p = r // _GRP, r % _GRP
                row, dbase = 2 * gi + p // 2, seq * (p % 2)
                if p % 2 == 1:
                    blk_v[row, pl.ds(88, 16)] = idx_v[r, pl.ds(0, 16)] >> 2
                for off in (0, 16, 32, 34):
                    blk_v[row, pl.ds(dbase + off, 16)] = (
                        idx_v[r, pl.ds(off, 16)] >> 2)

            fire(0, buf_a0, buf_a1, sem_a)
            fire(1, buf_b0, buf_b1, sem_b)

            def group_pair(kk, c2):
                ga = kk * 2
                drain(buf_a0, buf_a1, sem_a)

                @pl.when(kk > 0)
                def _():
                    drain_out(b0)

                extract(ga, buf_a0, buf_a1, 0)
                fire(jnp.minimum(ga + 2, _NGRP - 2), buf_a0, buf_a1, sem_a)
                drain(buf_b0, buf_b1, sem_b)
                extract(ga + 1, buf_b0, buf_b1, 1)
                fire(jnp.minimum(ga + 3, _NGRP - 1), buf_b0, buf_b1, sem_b)
                pltpu.async_copy(
                    ov, out_hbm.at[pl.ds(
                        pl.multiple_of(b0 + kk * 2 * _GRP, 2 * _GRP),
                        2 * _GRP)], sem_o)
                return c2

            lax.fori_loop(0, _NGRP // 2, group_pair, 0)
            drain(buf_a0, buf_a1, sem_a)
            drain(buf_b0, buf_b1, sem_b)
            drain_out(b0)
            return carry

        lax.fori_loop(0, nchunks, chunk, 0)

    return k(token_ids, table2)


def kernel(token_ids, weight):
    nblk = weight.shape[0] // 4
    table2 = (weight.T.reshape(32, nblk, 4)
              .transpose(1, 2, 0).reshape(nblk, 128))
    return _lookup(token_ids.astype(jnp.int32), table2)
